# baseline (speedup 1.0000x reference)
import functools

import jax
import jax.numpy as jnp
import numpy as np
from jax.experimental import pallas as pl
from jax.experimental.pallas import tpu as pltpu

N = 10000
NPAD = 10240
E = 160000
HID = 256
NA = 100
NB = 10
G = 64
NCONV = 4
CUTOFF = 10.0

BN = 1024
BE = 1280


def _f32(x):
    return x.astype(jnp.float32)


def _node_body(at_ref, rf_ref, pf_ref, bt_ref, t_ref, aemb_ref, afW_ref,
               inW_ref, inb_ref, out_ref):
    at = at_ref[:]
    oh_a = _f32(at == jax.lax.broadcasted_iota(jnp.int32, (1, NA), 1))
    ae = oh_a @ aemb_ref[:]
    afW = afW_ref[:]
    fr = rf_ref[:] @ afW
    fp = pf_ref[:] @ afW
    z1 = ae + fr
    z2 = fp - fr
    bt = bt_ref[:]
    oh_b = _f32(bt == jax.lax.broadcasted_iota(jnp.int32, (1, G), 1))
    inW = inW_ref[:]
    T64 = t_ref[:] * inW[HID:HID + 1, :]
    out_ref[:] = (z1 @ inW[0:HID // 2, :] + z2 @ inW[HID // 2:HID, :]
                  + oh_b @ T64 + inb_ref[:])


def _node_encode(atom_type, r_feat, p_feat, batch, t, atom_emb, atom_feat_W,
                 in_W, in_b):
    grid = NPAD // BN
    return pl.pallas_call(
        _node_body,
        grid=(grid,),
        in_specs=[
            pl.BlockSpec((BN, 1), lambda i: (i, 0)),
            pl.BlockSpec((BN, 28), lambda i: (i, 0)),
            pl.BlockSpec((BN, 28), lambda i: (i, 0)),
            pl.BlockSpec((BN, 1), lambda i: (i, 0)),
            pl.BlockSpec((G, 1), lambda i: (0, 0)),
            pl.BlockSpec((NA, HID // 2), lambda i: (0, 0)),
            pl.BlockSpec((28, HID // 2), lambda i: (0, 0)),
            pl.BlockSpec((HID + 1, HID), lambda i: (0, 0)),
            pl.BlockSpec((1, HID), lambda i: (0, 0)),
        ],
        out_specs=pl.BlockSpec((BN, HID), lambda i: (i, 0)),
        out_shape=jax.ShapeDtypeStruct((NPAD, HID), jnp.float32),
    )(atom_type, r_feat, p_feat, batch, t, atom_emb, atom_feat_W, in_W, in_b)


def _edge_attr_math(el, elT, tr, tp, eW1, eb1, eW2, eb2, bond, cW1, cb1,
                    cW2, cb2):
    pre = el * eW1[0:1, :] + elT * eW1[1:2, :] + eb1
    he = jnp.maximum(pre, 0.0) @ eW2 + eb2
    oh_r = _f32(tr == jax.lax.broadcasted_iota(jnp.int32, (1, NB), 1))
    oh_p = _f32(tp == jax.lax.broadcasted_iota(jnp.int32, (1, NB), 1))
    ar = he * (oh_r @ bond)
    ap = he * (oh_p @ bond)
    c1 = jnp.maximum(ar @ cW1[0:HID, :] + ap @ cW1[HID:2 * HID, :] + cb1, 0.0)
    return c1 @ cW2 + cb2


def _cur_edge_body(el2_ref, elT2_ref, tr_ref, tp_ref, eW1_ref, eb1_ref,
                   eW2_ref, eb2_ref, bond_ref, cW1_ref, cb1_ref, cW2_ref,
                   cb2_ref, We_ref, be_ref, out_ref):
    el = jnp.sqrt(el2_ref[:] + 1e-12)
    elT = jnp.sqrt(elT2_ref[:] + 1e-12)
    ea = _edge_attr_math(el, elT, tr_ref[:], tp_ref[:], eW1_ref[:],
                         eb1_ref[:], eW2_ref[:], eb2_ref[:], bond_ref[:],
                         cW1_ref[:], cb1_ref[:], cW2_ref[:], cb2_ref[:])
    Cw = 0.5 * (jnp.cos(el * (np.pi / CUTOFF)) + 1.0) * _f32(el < CUTOFF)
    for i in range(NCONV):
        out_ref[i] = (ea @ We_ref[i] + be_ref[i, 0:1, :]) * Cw


def _cur_edges(el2, elT2, tr, tp, e_W1, e_b1, e_W2, e_b2, bond_emb,
               cat_W1, cat_b1, cat_W2, cat_b2, conv_We, conv_be):
    grid = E // BE
    return pl.pallas_call(
        _cur_edge_body,
        grid=(grid,),
        in_specs=[
            pl.BlockSpec((BE, 1), lambda i: (i, 0)),
            pl.BlockSpec((BE, 1), lambda i: (i, 0)),
            pl.BlockSpec((BE, 1), lambda i: (i, 0)),
            pl.BlockSpec((BE, 1), lambda i: (i, 0)),
            pl.BlockSpec((2, HID), lambda i: (0, 0)),
            pl.BlockSpec((1, HID), lambda i: (0, 0)),
            pl.BlockSpec((HID, HID), lambda i: (0, 0)),
            pl.BlockSpec((1, HID), lambda i: (0, 0)),
            pl.BlockSpec((NB, HID), lambda i: (0, 0)),
            pl.BlockSpec((2 * HID, HID), lambda i: (0, 0)),
            pl.BlockSpec((1, HID), lambda i: (0, 0)),
            pl.BlockSpec((HID, HID), lambda i: (0, 0)),
            pl.BlockSpec((1, HID), lambda i: (0, 0)),
            pl.BlockSpec((NCONV, HID, HID), lambda i: (0, 0, 0)),
            pl.BlockSpec((NCONV, 1, HID), lambda i: (0, 0, 0)),
        ],
        out_specs=pl.BlockSpec((NCONV, BE, HID), lambda i: (0, i, 0)),
        out_shape=jax.ShapeDtypeStruct((NCONV, E, HID), jnp.float32),
    )(el2, elT2, tr, tp, e_W1, e_b1, e_W2, e_b2, bond_emb, cat_W1, cat_b1,
      cat_W2, cat_b2, conv_We, conv_be)


def _full_edge_body(el2_ref, elT2_ref, tr_ref, tp_ref, eW1_ref, eb1_ref,
                    eW2_ref, eb2_ref, bond_ref, cW1_ref, cb1_ref, cW2_ref,
                    cb2_ref, out_ref):
    el = jnp.sqrt(el2_ref[:] + 1e-12)
    elT = jnp.sqrt(elT2_ref[:] + 1e-12)
    out_ref[:] = _edge_attr_math(el, elT, tr_ref[:], tp_ref[:], eW1_ref[:],
                                 eb1_ref[:], eW2_ref[:], eb2_ref[:],
                                 bond_ref[:], cW1_ref[:], cb1_ref[:],
                                 cW2_ref[:], cb2_ref[:])


def _full_edges(el2, elT2, tr, tp, e_W1, e_b1, e_W2, e_b2, bond_emb,
                cat_W1, cat_b1, cat_W2, cat_b2):
    grid = E // BE
    return pl.pallas_call(
        _full_edge_body,
        grid=(grid,),
        in_specs=[
            pl.BlockSpec((BE, 1), lambda i: (i, 0)),
            pl.BlockSpec((BE, 1), lambda i: (i, 0)),
            pl.BlockSpec((BE, 1), lambda i: (i, 0)),
            pl.BlockSpec((BE, 1), lambda i: (i, 0)),
            pl.BlockSpec((2, HID), lambda i: (0, 0)),
            pl.BlockSpec((1, HID), lambda i: (0, 0)),
            pl.BlockSpec((HID, HID), lambda i: (0, 0)),
            pl.BlockSpec((1, HID), lambda i: (0, 0)),
            pl.BlockSpec((NB, HID), lambda i: (0, 0)),
            pl.BlockSpec((2 * HID, HID), lambda i: (0, 0)),
            pl.BlockSpec((1, HID), lambda i: (0, 0)),
            pl.BlockSpec((HID, HID), lambda i: (0, 0)),
            pl.BlockSpec((1, HID), lambda i: (0, 0)),
        ],
        out_specs=pl.BlockSpec((BE, HID), lambda i: (i, 0)),
        out_shape=jax.ShapeDtypeStruct((E, HID), jnp.float32),
    )(el2, elT2, tr, tp, e_W1, e_b1, e_W2, e_b2, bond_emb, cat_W1, cat_b1,
      cat_W2, cat_b2)


def _matmul_bias_body(h_ref, W_ref, b_ref, out_ref):
    out_ref[:] = h_ref[:] @ W_ref[:] + b_ref[:]


def _node_matmul(h, W, b):
    grid = NPAD // BN
    return pl.pallas_call(
        _matmul_bias_body,
        grid=(grid,),
        in_specs=[
            pl.BlockSpec((BN, HID), lambda i: (i, 0)),
            pl.BlockSpec((HID, HID), lambda i: (0, 0)),
            pl.BlockSpec((1, HID), lambda i: (0, 0)),
        ],
        out_specs=pl.BlockSpec((BN, HID), lambda i: (i, 0)),
        out_shape=jax.ShapeDtypeStruct((NPAD, HID), jnp.float32),
    )(h, W, b)


def _node_update_body(h_ref, agg_ref, W_ref, b_ref, out_ref):
    out_ref[:] = h_ref[:] + jnp.maximum(agg_ref[:] @ W_ref[:] + b_ref[:], 0.0)


def _node_update(h, agg, W, b):
    grid = NPAD // BN
    return pl.pallas_call(
        _node_update_body,
        grid=(grid,),
        in_specs=[
            pl.BlockSpec((BN, HID), lambda i: (i, 0)),
            pl.BlockSpec((BN, HID), lambda i: (i, 0)),
            pl.BlockSpec((HID, HID), lambda i: (0, 0)),
            pl.BlockSpec((1, HID), lambda i: (0, 0)),
        ],
        out_specs=pl.BlockSpec((BN, HID), lambda i: (i, 0)),
        out_shape=jax.ShapeDtypeStruct((NPAD, HID), jnp.float32),
    )(h, agg, W, b)


def _score_body(hh_ref, fe_ref, W1_ref, b1_ref, W2_ref, b2_ref, W3_ref,
                b3_ref, out_ref):
    W1 = W1_ref[:]
    p = jnp.maximum(hh_ref[:] @ W1[0:HID, :] + fe_ref[:] @ W1[HID:2 * HID, :]
                    + b1_ref[:], 0.0)
    p = jnp.maximum(p @ W2_ref[:] + b2_ref[:], 0.0)
    out_ref[:] = p @ W3_ref[:] + b3_ref[:]


def _score(hh, fedge, s_W1, s_b1, s_W2, s_b2, s_W3, s_b3):
    grid = E // BE
    return pl.pallas_call(
        _score_body,
        grid=(grid,),
        in_specs=[
            pl.BlockSpec((BE, HID), lambda i: (i, 0)),
            pl.BlockSpec((BE, HID), lambda i: (i, 0)),
            pl.BlockSpec((2 * HID, HID), lambda i: (0, 0)),
            pl.BlockSpec((1, HID), lambda i: (0, 0)),
            pl.BlockSpec((HID, HID // 2), lambda i: (0, 0)),
            pl.BlockSpec((1, HID // 2), lambda i: (0, 0)),
            pl.BlockSpec((HID // 2, 1), lambda i: (0, 0)),
            pl.BlockSpec((1, 1), lambda i: (0, 0)),
        ],
        out_specs=pl.BlockSpec((BE, 1), lambda i: (i, 0)),
        out_shape=jax.ShapeDtypeStruct((E, 1), jnp.float32),
    )(hh, fedge, s_W1, s_b1, s_W2, s_b2, s_W3, s_b3)


def _sqdist(p, i0, i1):
    d = p[i0] - p[i1]
    return jnp.sum(d * d, axis=-1, keepdims=True)


def kernel(atom_type, r_feat, p_feat, t, pos, pos_init, batch,
           current_edge_index, current_edge_feat_r, current_edge_feat_p,
           full_edge_index, full_type_r, full_type_p,
           atom_emb, atom_feat_W, bond_emb,
           e_W1, e_b1, e_W2, e_b2,
           cat_W1, cat_b1, cat_W2, cat_b2,
           in_W, in_b,
           conv_Wn, conv_bn, conv_We, conv_be, conv_Wu, conv_bu,
           s_W1, s_b1, s_W2, s_b2, s_W3, s_b3):
    pad_n = NPAD - N
    at2 = jnp.pad(atom_type.astype(jnp.int32), (0, pad_n))[:, None]
    bt2 = jnp.pad(batch.astype(jnp.int32), (0, pad_n))[:, None]
    rf2 = jnp.pad(r_feat, ((0, pad_n), (0, 0)))
    pf2 = jnp.pad(p_feat, ((0, pad_n), (0, 0)))

    h = _node_encode(at2, rf2, pf2, bt2, t[:, None], atom_emb, atom_feat_W,
                     in_W, in_b[None, :])

    cei0 = current_edge_index[0].astype(jnp.int32)
    cei1 = current_edge_index[1].astype(jnp.int32)
    fei0 = full_edge_index[0].astype(jnp.int32)
    fei1 = full_edge_index[1].astype(jnp.int32)

    el2 = _sqdist(pos, cei0, cei1)
    elT2 = _sqdist(pos_init, cei0, cei1)
    fl2 = _sqdist(pos, fei0, fei1)
    flT2 = _sqdist(pos_init, fei0, fei1)

    en_all = _cur_edges(el2, elT2,
                        current_edge_feat_r.astype(jnp.int32)[:, None],
                        current_edge_feat_p.astype(jnp.int32)[:, None],
                        e_W1, e_b1[None, :], e_W2, e_b2[None, :], bond_emb,
                        cat_W1, cat_b1[None, :], cat_W2, cat_b2[None, :],
                        conv_We, conv_be[:, None, :])

    fedge = _full_edges(fl2, flT2,
                        full_type_r.astype(jnp.int32)[:, None],
                        full_type_p.astype(jnp.int32)[:, None],
                        e_W1, e_b1[None, :], e_W2, e_b2[None, :], bond_emb,
                        cat_W1, cat_b1[None, :], cat_W2, cat_b2[None, :])

    for i in range(NCONV):
        hn = _node_matmul(h, conv_Wn[i], conv_bn[i][None, :])
        m = hn[cei0] * en_all[i]
        agg = jax.ops.segment_sum(m, cei1, num_segments=NPAD)
        h = _node_update(h, agg, conv_Wu[i], conv_bu[i][None, :])

    hh = h[fei0] * h[fei1]
    return _score(hh, fedge, s_W1, s_b1[None, :], s_W2, s_b2[None, :],
                  s_W3, s_b3[None, :])


# speedup vs baseline: 2.6948x; 2.6948x over previous
import functools

import jax
import jax.numpy as jnp
import numpy as np
from jax import lax
from jax.experimental import pallas as pl
from jax.experimental.pallas import tpu as pltpu
from jax.experimental.pallas import tpu_sc as plsc

N = 10000
NPAD = 10240
E = 160000
HID = 256
NA = 100
NB = 10
G = 64
NCONV = 4
CUTOFF = 10.0

BN = 1024
BE = 1280


def _f32(x):
    return x.astype(jnp.float32)


def _node_body(at_ref, rf_ref, pf_ref, bt_ref, t_ref, aemb_ref, afW_ref,
               inW_ref, inb_ref, out_ref):
    at = at_ref[:]
    oh_a = _f32(at == jax.lax.broadcasted_iota(jnp.int32, (1, NA), 1))
    ae = oh_a @ aemb_ref[:]
    afW = afW_ref[:]
    fr = rf_ref[:] @ afW
    fp = pf_ref[:] @ afW
    z1 = ae + fr
    z2 = fp - fr
    bt = bt_ref[:]
    oh_b = _f32(bt == jax.lax.broadcasted_iota(jnp.int32, (1, G), 1))
    inW = inW_ref[:]
    T64 = t_ref[:] * inW[HID:HID + 1, :]
    out_ref[:] = (z1 @ inW[0:HID // 2, :] + z2 @ inW[HID // 2:HID, :]
                  + oh_b @ T64 + inb_ref[:])


def _node_encode(atom_type, r_feat, p_feat, batch, t, atom_emb, atom_feat_W,
                 in_W, in_b):
    grid = NPAD // BN
    return pl.pallas_call(
        _node_body,
        grid=(grid,),
        in_specs=[
            pl.BlockSpec((BN, 1), lambda i: (i, 0)),
            pl.BlockSpec((BN, 28), lambda i: (i, 0)),
            pl.BlockSpec((BN, 28), lambda i: (i, 0)),
            pl.BlockSpec((BN, 1), lambda i: (i, 0)),
            pl.BlockSpec((G, 1), lambda i: (0, 0)),
            pl.BlockSpec((NA, HID // 2), lambda i: (0, 0)),
            pl.BlockSpec((28, HID // 2), lambda i: (0, 0)),
            pl.BlockSpec((HID + 1, HID), lambda i: (0, 0)),
            pl.BlockSpec((1, HID), lambda i: (0, 0)),
        ],
        out_specs=pl.BlockSpec((BN, HID), lambda i: (i, 0)),
        out_shape=jax.ShapeDtypeStruct((NPAD, HID), jnp.float32),
    )(atom_type, r_feat, p_feat, batch, t, atom_emb, atom_feat_W, in_W, in_b)


def _edge_attr_math(el, elT, tr, tp, eW1, eb1, eW2, eb2, bond, cW1, cb1,
                    cW2, cb2):
    pre = el * eW1[0:1, :] + elT * eW1[1:2, :] + eb1
    he = jnp.maximum(pre, 0.0) @ eW2 + eb2
    oh_r = _f32(tr == jax.lax.broadcasted_iota(jnp.int32, (1, NB), 1))
    oh_p = _f32(tp == jax.lax.broadcasted_iota(jnp.int32, (1, NB), 1))
    ar = he * (oh_r @ bond)
    ap = he * (oh_p @ bond)
    c1 = jnp.maximum(ar @ cW1[0:HID, :] + ap @ cW1[HID:2 * HID, :] + cb1, 0.0)
    return c1 @ cW2 + cb2


def _cur_edge_body(el2_ref, elT2_ref, tr_ref, tp_ref, eW1_ref, eb1_ref,
                   eW2_ref, eb2_ref, bond_ref, cW1_ref, cb1_ref, cW2_ref,
                   cb2_ref, We_ref, be_ref, out_ref):
    el = jnp.sqrt(el2_ref[:] + 1e-12)
    elT = jnp.sqrt(elT2_ref[:] + 1e-12)
    ea = _edge_attr_math(el, elT, tr_ref[:], tp_ref[:], eW1_ref[:],
                         eb1_ref[:], eW2_ref[:], eb2_ref[:], bond_ref[:],
                         cW1_ref[:], cb1_ref[:], cW2_ref[:], cb2_ref[:])
    Cw = 0.5 * (jnp.cos(el * (np.pi / CUTOFF)) + 1.0) * _f32(el < CUTOFF)
    for i in range(NCONV):
        en = (ea @ We_ref[i] + be_ref[i, 0:1, :]) * Cw
        out_ref[i, 0] = en[:, 0:HID // 2]
        out_ref[i, 1] = en[:, HID // 2:HID]


def _cur_edges(el2, elT2, tr, tp, e_W1, e_b1, e_W2, e_b2, bond_emb,
               cat_W1, cat_b1, cat_W2, cat_b2, conv_We, conv_be):
    grid = E // BE
    return pl.pallas_call(
        _cur_edge_body,
        grid=(grid,),
        in_specs=[
            pl.BlockSpec((BE, 1), lambda i: (i, 0)),
            pl.BlockSpec((BE, 1), lambda i: (i, 0)),
            pl.BlockSpec((BE, 1), lambda i: (i, 0)),
            pl.BlockSpec((BE, 1), lambda i: (i, 0)),
            pl.BlockSpec((2, HID), lambda i: (0, 0)),
            pl.BlockSpec((1, HID), lambda i: (0, 0)),
            pl.BlockSpec((HID, HID), lambda i: (0, 0)),
            pl.BlockSpec((1, HID), lambda i: (0, 0)),
            pl.BlockSpec((NB, HID), lambda i: (0, 0)),
            pl.BlockSpec((2 * HID, HID), lambda i: (0, 0)),
            pl.BlockSpec((1, HID), lambda i: (0, 0)),
            pl.BlockSpec((HID, HID), lambda i: (0, 0)),
            pl.BlockSpec((1, HID), lambda i: (0, 0)),
            pl.BlockSpec((NCONV, HID, HID), lambda i: (0, 0, 0)),
            pl.BlockSpec((NCONV, 1, HID), lambda i: (0, 0, 0)),
        ],
        out_specs=pl.BlockSpec((NCONV, 2, BE, HID // 2),
                               lambda i: (0, 0, i, 0)),
        out_shape=jax.ShapeDtypeStruct((NCONV, 2, E, HID // 2), jnp.float32),
    )(el2, elT2, tr, tp, e_W1, e_b1, e_W2, e_b2, bond_emb, cat_W1, cat_b1,
      cat_W2, cat_b2, conv_We, conv_be)


def _full_edge_body(el2_ref, elT2_ref, tr_ref, tp_ref, eW1_ref, eb1_ref,
                    eW2_ref, eb2_ref, bond_ref, cW1_ref, cb1_ref, cW2_ref,
                    cb2_ref, out_ref):
    el = jnp.sqrt(el2_ref[:] + 1e-12)
    elT = jnp.sqrt(elT2_ref[:] + 1e-12)
    out_ref[:] = _edge_attr_math(el, elT, tr_ref[:], tp_ref[:], eW1_ref[:],
                                 eb1_ref[:], eW2_ref[:], eb2_ref[:],
                                 bond_ref[:], cW1_ref[:], cb1_ref[:],
                                 cW2_ref[:], cb2_ref[:])


def _full_edges(el2, elT2, tr, tp, e_W1, e_b1, e_W2, e_b2, bond_emb,
                cat_W1, cat_b1, cat_W2, cat_b2):
    grid = E // BE
    return pl.pallas_call(
        _full_edge_body,
        grid=(grid,),
        in_specs=[
            pl.BlockSpec((BE, 1), lambda i: (i, 0)),
            pl.BlockSpec((BE, 1), lambda i: (i, 0)),
            pl.BlockSpec((BE, 1), lambda i: (i, 0)),
            pl.BlockSpec((BE, 1), lambda i: (i, 0)),
            pl.BlockSpec((2, HID), lambda i: (0, 0)),
            pl.BlockSpec((1, HID), lambda i: (0, 0)),
            pl.BlockSpec((HID, HID), lambda i: (0, 0)),
            pl.BlockSpec((1, HID), lambda i: (0, 0)),
            pl.BlockSpec((NB, HID), lambda i: (0, 0)),
            pl.BlockSpec((2 * HID, HID), lambda i: (0, 0)),
            pl.BlockSpec((1, HID), lambda i: (0, 0)),
            pl.BlockSpec((HID, HID), lambda i: (0, 0)),
            pl.BlockSpec((1, HID), lambda i: (0, 0)),
        ],
        out_specs=pl.BlockSpec((BE, HID), lambda i: (i, 0)),
        out_shape=jax.ShapeDtypeStruct((E, HID), jnp.float32),
    )(el2, elT2, tr, tp, e_W1, e_b1, e_W2, e_b2, bond_emb, cat_W1, cat_b1,
      cat_W2, cat_b2)


def _matmul_bias_body(h_ref, W_ref, b_ref, out_ref):
    res = h_ref[:] @ W_ref[:] + b_ref[:]
    out_ref[0] = res[:, 0:HID // 2]
    out_ref[1] = res[:, HID // 2:HID]


def _node_matmul(h, W, b):
    grid = NPAD // BN
    return pl.pallas_call(
        _matmul_bias_body,
        grid=(grid,),
        in_specs=[
            pl.BlockSpec((BN, HID), lambda i: (i, 0)),
            pl.BlockSpec((HID, HID), lambda i: (0, 0)),
            pl.BlockSpec((1, HID), lambda i: (0, 0)),
        ],
        out_specs=pl.BlockSpec((2, BN, HID // 2), lambda i: (0, i, 0)),
        out_shape=jax.ShapeDtypeStruct((2, NPAD, HID // 2), jnp.float32),
    )(h, W, b)


def _node_update_body(stacked, h_ref, agg_ref, W_ref, b_ref, out_ref):
    W = W_ref[:]
    upd = agg_ref[0] @ W[0:HID // 2, :] + agg_ref[1] @ W[HID // 2:HID, :]
    res = h_ref[:] + jnp.maximum(upd + b_ref[:], 0.0)
    if stacked:
        out_ref[0] = res[:, 0:HID // 2]
        out_ref[1] = res[:, HID // 2:HID]
    else:
        out_ref[:] = res


def _node_update(h, agg, W, b, stacked=False):
    grid = NPAD // BN
    if stacked:
        out_spec = pl.BlockSpec((2, BN, HID // 2), lambda i: (0, i, 0))
        out_shape = jax.ShapeDtypeStruct((2, NPAD, HID // 2), jnp.float32)
    else:
        out_spec = pl.BlockSpec((BN, HID), lambda i: (i, 0))
        out_shape = jax.ShapeDtypeStruct((NPAD, HID), jnp.float32)
    return pl.pallas_call(
        functools.partial(_node_update_body, stacked),
        grid=(grid,),
        in_specs=[
            pl.BlockSpec((BN, HID), lambda i: (i, 0)),
            pl.BlockSpec((2, BN, HID // 2), lambda i: (0, i, 0)),
            pl.BlockSpec((HID, HID), lambda i: (0, 0)),
            pl.BlockSpec((1, HID), lambda i: (0, 0)),
        ],
        out_specs=out_spec,
        out_shape=out_shape,
    )(h, agg, W, b)


def _score_body(hh_ref, fe_ref, W1_ref, b1_ref, W2_ref, b2_ref, W3_ref,
                b3_ref, out_ref):
    W1 = W1_ref[:]
    p = jnp.maximum(hh_ref[0] @ W1[0:HID // 2, :]
                    + hh_ref[1] @ W1[HID // 2:HID, :]
                    + fe_ref[:] @ W1[HID:2 * HID, :]
                    + b1_ref[:], 0.0)
    p = jnp.maximum(p @ W2_ref[:] + b2_ref[:], 0.0)
    out_ref[:] = p @ W3_ref[:] + b3_ref[:]


def _score(hh, fedge, s_W1, s_b1, s_W2, s_b2, s_W3, s_b3):
    grid = E // BE
    return pl.pallas_call(
        _score_body,
        grid=(grid,),
        in_specs=[
            pl.BlockSpec((2, BE, HID // 2), lambda i: (0, i, 0)),
            pl.BlockSpec((BE, HID), lambda i: (i, 0)),
            pl.BlockSpec((2 * HID, HID), lambda i: (0, 0)),
            pl.BlockSpec((1, HID), lambda i: (0, 0)),
            pl.BlockSpec((HID, HID // 2), lambda i: (0, 0)),
            pl.BlockSpec((1, HID // 2), lambda i: (0, 0)),
            pl.BlockSpec((HID // 2, 1), lambda i: (0, 0)),
            pl.BlockSpec((1, 1), lambda i: (0, 0)),
        ],
        out_specs=pl.BlockSpec((BE, 1), lambda i: (i, 0)),
        out_shape=jax.ShapeDtypeStruct((E, 1), jnp.float32),
    )(hh, fedge, s_W1, s_b1, s_W2, s_b2, s_W3, s_b3)


_HH = HID // 2
_CK = 128
_NCHUNK = E // _CK
_CPW = (_NCHUNK + 15) // 16
_ROWS_PER_TILE = 632
_AGG_ROWS = 16 * _ROWS_PER_TILE


def _zero_fill(zb):
    @pl.loop(0, _CK)
    def _(r):
        z = jnp.zeros((16,), jnp.float32)
        for cg in range(_HH // 16):
            zb[r, pl.ds(cg * 16, 16)] = z


def _mul_inplace(a, b):
    @pl.loop(0, _CK)
    def _(r):
        for cg in range(_HH // 16):
            sl = pl.ds(cg * 16, 16)
            a[r, sl] = a[r, sl] * b[r, sl]


@functools.lru_cache(maxsize=None)
def _make_conv_sc(conv_i):
    I = conv_i

    @functools.partial(
        pl.kernel,
        mesh=plsc.VectorSubcoreMesh(core_axis_name="c", subcore_axis_name="s"),
        out_type=jax.ShapeDtypeStruct((2 * NPAD, _HH), jnp.float32),
        scratch_types=[
            pltpu.VMEM((_CK,), jnp.int32),
            pltpu.VMEM((_CK,), jnp.int32),
            pltpu.VMEM((_CK, _HH), jnp.float32),
            pltpu.VMEM((_CK, _HH), jnp.float32),
            pltpu.VMEM((_CK, _HH), jnp.float32),
            pltpu.VMEM_SHARED((_AGG_ROWS, _HH), jnp.float32),
            pltpu.SemaphoreType.DMA,
        ],
    )
    def conv_sc(hn2, en, src2, dst, out, idx_s, idx_d, gath, enb, zb,
                aggs, sem):
        c = lax.axis_index("c")
        s = lax.axis_index("s")
        _zero_fill(zb)
        zrow = s * _ROWS_PER_TILE
        for k in range(4):
            pltpu.sync_copy(zb, aggs.at[pl.ds(zrow + k * _CK, _CK), :])
        pltpu.sync_copy(zb.at[pl.ds(0, _ROWS_PER_TILE - 4 * _CK), :],
                        aggs.at[pl.ds(zrow + 4 * _CK,
                                      _ROWS_PER_TILE - 4 * _CK), :])
        plsc.subcore_barrier()

        @pl.loop(0, _CPW)
        def _(g):
            q = g * 16 + s

            @pl.when(q < _NCHUNK)
            def _():
                base = q * _CK
                pltpu.sync_copy(src2.at[pl.ds(c * E + base, _CK)], idx_s)
                pltpu.sync_copy(dst.at[pl.ds(base, _CK)], idx_d)
                pltpu.async_copy(hn2.at[idx_s], gath, sem).wait()
                pltpu.sync_copy(en.at[I, pl.ds(c * E + base, _CK), :], enb)
                _mul_inplace(gath, enb)
                pltpu.sync_copy(gath, aggs.at[idx_d], add=True)

        plsc.subcore_barrier()
        pltpu.sync_copy(aggs.at[pl.ds(zrow, _ROWS_PER_TILE), :],
                        out.at[pl.ds(c * NPAD + zrow, _ROWS_PER_TILE), :])

    return conv_sc


@functools.lru_cache(maxsize=None)
def _make_pair_sc():
    @functools.partial(
        pl.kernel,
        mesh=plsc.VectorSubcoreMesh(core_axis_name="c", subcore_axis_name="s"),
        out_type=jax.ShapeDtypeStruct((2 * E, _HH), jnp.float32),
        scratch_types=[
            pltpu.VMEM((_CK,), jnp.int32),
            pltpu.VMEM((_CK,), jnp.int32),
            pltpu.VMEM((_CK, _HH), jnp.float32),
            pltpu.VMEM((_CK, _HH), jnp.float32),
            pltpu.SemaphoreType.DMA,
        ],
    )
    def pair_sc(h2, ia2, ib2, out, idx_a, idx_b, ga, gb, sem):
        c = lax.axis_index("c")
        s = lax.axis_index("s")

        @pl.loop(0, _CPW)
        def _(g):
            q = g * 16 + s

            @pl.when(q < _NCHUNK)
            def _():
                base = q * _CK
                pltpu.sync_copy(ia2.at[pl.ds(c * E + base, _CK)], idx_a)
                pltpu.sync_copy(ib2.at[pl.ds(c * E + base, _CK)], idx_b)
                pltpu.async_copy(h2.at[idx_a], ga, sem).wait()
                pltpu.async_copy(h2.at[idx_b], gb, sem).wait()
                _mul_inplace(ga, gb)
                pltpu.sync_copy(ga, out.at[pl.ds(c * E + base, _CK), :])

    return pair_sc


_DW = (2 * E) // 32
_DCK = 2000
_DG = _DCK // 16


@functools.lru_cache(maxsize=None)
def _make_dist_sc():
    @functools.partial(
        pl.kernel,
        mesh=plsc.VectorSubcoreMesh(core_axis_name="c", subcore_axis_name="s"),
        compiler_params=pltpu.CompilerParams(needs_layout_passes=False),
        out_type=[jax.ShapeDtypeStruct((2 * E,), jnp.float32),
                  jax.ShapeDtypeStruct((2 * E,), jnp.float32)],
        scratch_types=[
            [pltpu.VMEM((N,), jnp.float32) for _ in range(6)],
            pltpu.VMEM((_DCK,), jnp.int32),
            pltpu.VMEM((_DCK,), jnp.int32),
            pltpu.VMEM((_DCK,), jnp.float32),
            pltpu.VMEM((_DCK,), jnp.float32),
        ],
    )
    def dist_sc(px, py, pz, qx, qy, qz, i0, i1, d2a, d2b, coords, iv0, iv1,
                oa, ob):
        c = lax.axis_index("c")
        s = lax.axis_index("s")
        w = s * 2 + c
        for src_hbm, dstv in zip((px, py, pz, qx, qy, qz), coords):
            pltpu.sync_copy(src_hbm, dstv)
        base = w * _DW
        for k in range(_DW // _DCK):
            cbase = base + k * _DCK
            pltpu.sync_copy(i0.at[pl.ds(cbase, _DCK)], iv0)
            pltpu.sync_copy(i1.at[pl.ds(cbase, _DCK)], iv1)

            @pl.loop(0, _DG)
            def _(j):
                sl = pl.ds(j * 16, 16)
                a0 = iv0[sl]
                a1 = iv1[sl]
                dx = plsc.load_gather(coords[0], [a0]) - plsc.load_gather(coords[0], [a1])
                dy = plsc.load_gather(coords[1], [a0]) - plsc.load_gather(coords[1], [a1])
                dz = plsc.load_gather(coords[2], [a0]) - plsc.load_gather(coords[2], [a1])
                oa[sl] = dx * dx + dy * dy + dz * dz
                ex = plsc.load_gather(coords[3], [a0]) - plsc.load_gather(coords[3], [a1])
                ey = plsc.load_gather(coords[4], [a0]) - plsc.load_gather(coords[4], [a1])
                ez = plsc.load_gather(coords[5], [a0]) - plsc.load_gather(coords[5], [a1])
                ob[sl] = ex * ex + ey * ey + ez * ez

            pltpu.sync_copy(oa, d2a.at[pl.ds(cbase, _DCK)])
            pltpu.sync_copy(ob, d2b.at[pl.ds(cbase, _DCK)])

    return dist_sc


def _sqdist(p, i0, i1):
    d = p[i0] - p[i1]
    return jnp.sum(d * d, axis=-1, keepdims=True)


def kernel(atom_type, r_feat, p_feat, t, pos, pos_init, batch,
           current_edge_index, current_edge_feat_r, current_edge_feat_p,
           full_edge_index, full_type_r, full_type_p,
           atom_emb, atom_feat_W, bond_emb,
           e_W1, e_b1, e_W2, e_b2,
           cat_W1, cat_b1, cat_W2, cat_b2,
           in_W, in_b,
           conv_Wn, conv_bn, conv_We, conv_be, conv_Wu, conv_bu,
           s_W1, s_b1, s_W2, s_b2, s_W3, s_b3):
    pad_n = NPAD - N
    at2 = jnp.pad(atom_type.astype(jnp.int32), (0, pad_n))[:, None]
    bt2 = jnp.pad(batch.astype(jnp.int32), (0, pad_n))[:, None]
    rf2 = jnp.pad(r_feat, ((0, pad_n), (0, 0)))
    pf2 = jnp.pad(p_feat, ((0, pad_n), (0, 0)))

    h = _node_encode(at2, rf2, pf2, bt2, t[:, None], atom_emb, atom_feat_W,
                     in_W, in_b[None, :])

    cei0 = current_edge_index[0].astype(jnp.int32)
    cei1 = current_edge_index[1].astype(jnp.int32)
    fei0 = full_edge_index[0].astype(jnp.int32)
    fei1 = full_edge_index[1].astype(jnp.int32)

    i0cat = jnp.concatenate([cei0, fei0])
    i1cat = jnp.concatenate([cei1, fei1])
    d2a, d2b = _make_dist_sc()(pos[:, 0], pos[:, 1], pos[:, 2],
                               pos_init[:, 0], pos_init[:, 1], pos_init[:, 2],
                               i0cat, i1cat)
    el2 = d2a[:E][:, None]
    fl2 = d2a[E:][:, None]
    elT2 = d2b[:E][:, None]
    flT2 = d2b[E:][:, None]

    en_all = _cur_edges(el2, elT2,
                        current_edge_feat_r.astype(jnp.int32)[:, None],
                        current_edge_feat_p.astype(jnp.int32)[:, None],
                        e_W1, e_b1[None, :], e_W2, e_b2[None, :], bond_emb,
                        cat_W1, cat_b1[None, :], cat_W2, cat_b2[None, :],
                        conv_We, conv_be[:, None, :])

    fedge = _full_edges(fl2, flT2,
                        full_type_r.astype(jnp.int32)[:, None],
                        full_type_p.astype(jnp.int32)[:, None],
                        e_W1, e_b1[None, :], e_W2, e_b2[None, :], bond_emb,
                        cat_W1, cat_b1[None, :], cat_W2, cat_b2[None, :])

    en_flat = en_all.reshape(NCONV, 2 * E, HID // 2)
    src2 = jnp.concatenate([cei0, cei0 + NPAD])
    for i in range(NCONV):
        hn = _node_matmul(h, conv_Wn[i], conv_bn[i][None, :])
        hn2 = hn.reshape(2 * NPAD, HID // 2)
        agg = _make_conv_sc(i)(hn2, en_flat, src2, cei1)
        h = _node_update(h, agg.reshape(2, NPAD, HID // 2),
                         conv_Wu[i], conv_bu[i][None, :],
                         stacked=(i == NCONV - 1))

    h2 = h.reshape(2 * NPAD, HID // 2)
    ia2 = jnp.concatenate([fei0, fei0 + NPAD])
    ib2 = jnp.concatenate([fei1, fei1 + NPAD])
    hh = _make_pair_sc()(h2, ia2, ib2).reshape(2, E, HID // 2)
    return _score(hh, fedge, s_W1, s_b1[None, :], s_W2, s_b2[None, :],
                  s_W3, s_b3[None, :])


# speedup vs baseline: 2.8414x; 1.0544x over previous
import functools

import jax
import jax.numpy as jnp
import numpy as np
from jax import lax
from jax.experimental import pallas as pl
from jax.experimental.pallas import tpu as pltpu
from jax.experimental.pallas import tpu_sc as plsc

N = 10000
NPAD = 10240
E = 160000
EPAD = 163840
HID = 256
NA = 100
NB = 10
G = 64
NCONV = 4
CUTOFF = 10.0

BN = 1024
BE = 1280


def _f32(x):
    return x.astype(jnp.float32)


def _node_body(at_ref, rf_ref, pf_ref, bt_ref, t_ref, aemb_ref, afW_ref,
               inW_ref, inb_ref, out_ref):
    at = at_ref[:]
    oh_a = _f32(at == jax.lax.broadcasted_iota(jnp.int32, (1, NA), 1))
    ae = oh_a @ aemb_ref[:]
    afW = afW_ref[:]
    fr = rf_ref[:] @ afW
    fp = pf_ref[:] @ afW
    z1 = ae + fr
    z2 = fp - fr
    bt = bt_ref[:]
    oh_b = _f32(bt == jax.lax.broadcasted_iota(jnp.int32, (1, G), 1))
    inW = inW_ref[:]
    T64 = t_ref[:] * inW[HID:HID + 1, :]
    out_ref[:] = (z1 @ inW[0:HID // 2, :] + z2 @ inW[HID // 2:HID, :]
                  + oh_b @ T64 + inb_ref[:])


def _node_encode(atom_type, r_feat, p_feat, batch, t, atom_emb, atom_feat_W,
                 in_W, in_b):
    grid = NPAD // BN
    return pl.pallas_call(
        _node_body,
        grid=(grid,),
        in_specs=[
            pl.BlockSpec((BN, 1), lambda i: (i, 0)),
            pl.BlockSpec((BN, 28), lambda i: (i, 0)),
            pl.BlockSpec((BN, 28), lambda i: (i, 0)),
            pl.BlockSpec((BN, 1), lambda i: (i, 0)),
            pl.BlockSpec((G, 1), lambda i: (0, 0)),
            pl.BlockSpec((NA, HID // 2), lambda i: (0, 0)),
            pl.BlockSpec((28, HID // 2), lambda i: (0, 0)),
            pl.BlockSpec((HID + 1, HID), lambda i: (0, 0)),
            pl.BlockSpec((1, HID), lambda i: (0, 0)),
        ],
        out_specs=pl.BlockSpec((BN, HID), lambda i: (i, 0)),
        out_shape=jax.ShapeDtypeStruct((NPAD, HID), jnp.float32),
    )(atom_type, r_feat, p_feat, batch, t, atom_emb, atom_feat_W, in_W, in_b)


def _edge_attr_math(el, elT, tr, tp, eW1, eb1, eW2, eb2, bond, cW1, cb1,
                    cW2, cb2):
    pre = el * eW1[0:1, :] + elT * eW1[1:2, :] + eb1
    he = jnp.maximum(pre, 0.0) @ eW2 + eb2
    oh_r = _f32(tr == jax.lax.broadcasted_iota(jnp.int32, (1, NB), 1))
    oh_p = _f32(tp == jax.lax.broadcasted_iota(jnp.int32, (1, NB), 1))
    ar = he * (oh_r @ bond)
    ap = he * (oh_p @ bond)
    c1 = jnp.maximum(ar @ cW1[0:HID, :] + ap @ cW1[HID:2 * HID, :] + cb1, 0.0)
    return c1 @ cW2 + cb2


def _cur_edge_body(el2_ref, elT2_ref, tr_ref, tp_ref, eW1_ref, eb1_ref,
                   eW2_ref, eb2_ref, bond_ref, cW1_ref, cb1_ref, cW2_ref,
                   cb2_ref, We_ref, be_ref, out_ref):
    el = jnp.sqrt(el2_ref[:] + 1e-12)
    elT = jnp.sqrt(elT2_ref[:] + 1e-12)
    ea = _edge_attr_math(el, elT, tr_ref[:], tp_ref[:], eW1_ref[:],
                         eb1_ref[:], eW2_ref[:], eb2_ref[:], bond_ref[:],
                         cW1_ref[:], cb1_ref[:], cW2_ref[:], cb2_ref[:])
    Cw = 0.5 * (jnp.cos(el * (np.pi / CUTOFF)) + 1.0) * _f32(el < CUTOFF)
    for i in range(NCONV):
        en = (ea @ We_ref[i] + be_ref[i, 0:1, :]) * Cw
        out_ref[i, 0] = en[:, 0:HID // 2]
        out_ref[i, 1] = en[:, HID // 2:HID]


def _cur_edges(el2, elT2, tr, tp, e_W1, e_b1, e_W2, e_b2, bond_emb,
               cat_W1, cat_b1, cat_W2, cat_b2, conv_We, conv_be):
    grid = EPAD // BE
    return pl.pallas_call(
        _cur_edge_body,
        grid=(grid,),
        in_specs=[
            pl.BlockSpec((BE, 1), lambda i: (i, 0)),
            pl.BlockSpec((BE, 1), lambda i: (i, 0)),
            pl.BlockSpec((BE, 1), lambda i: (i, 0)),
            pl.BlockSpec((BE, 1), lambda i: (i, 0)),
            pl.BlockSpec((2, HID), lambda i: (0, 0)),
            pl.BlockSpec((1, HID), lambda i: (0, 0)),
            pl.BlockSpec((HID, HID), lambda i: (0, 0)),
            pl.BlockSpec((1, HID), lambda i: (0, 0)),
            pl.BlockSpec((NB, HID), lambda i: (0, 0)),
            pl.BlockSpec((2 * HID, HID), lambda i: (0, 0)),
            pl.BlockSpec((1, HID), lambda i: (0, 0)),
            pl.BlockSpec((HID, HID), lambda i: (0, 0)),
            pl.BlockSpec((1, HID), lambda i: (0, 0)),
            pl.BlockSpec((NCONV, HID, HID), lambda i: (0, 0, 0)),
            pl.BlockSpec((NCONV, 1, HID), lambda i: (0, 0, 0)),
        ],
        out_specs=pl.BlockSpec((NCONV, 2, BE, HID // 2),
                               lambda i: (0, 0, i, 0)),
        out_shape=jax.ShapeDtypeStruct((NCONV, 2, EPAD, HID // 2),
                                       jnp.float32),
    )(el2, elT2, tr, tp, e_W1, e_b1, e_W2, e_b2, bond_emb, cat_W1, cat_b1,
      cat_W2, cat_b2, conv_We, conv_be)


def _full_edge_body(el2_ref, elT2_ref, tr_ref, tp_ref, eW1_ref, eb1_ref,
                    eW2_ref, eb2_ref, bond_ref, cW1_ref, cb1_ref, cW2_ref,
                    cb2_ref, out_ref):
    el = jnp.sqrt(el2_ref[:] + 1e-12)
    elT = jnp.sqrt(elT2_ref[:] + 1e-12)
    out_ref[:] = _edge_attr_math(el, elT, tr_ref[:], tp_ref[:], eW1_ref[:],
                                 eb1_ref[:], eW2_ref[:], eb2_ref[:],
                                 bond_ref[:], cW1_ref[:], cb1_ref[:],
                                 cW2_ref[:], cb2_ref[:])


def _full_edges(el2, elT2, tr, tp, e_W1, e_b1, e_W2, e_b2, bond_emb,
                cat_W1, cat_b1, cat_W2, cat_b2):
    grid = EPAD // BE
    return pl.pallas_call(
        _full_edge_body,
        grid=(grid,),
        in_specs=[
            pl.BlockSpec((BE, 1), lambda i: (i, 0)),
            pl.BlockSpec((BE, 1), lambda i: (i, 0)),
            pl.BlockSpec((BE, 1), lambda i: (i, 0)),
            pl.BlockSpec((BE, 1), lambda i: (i, 0)),
            pl.BlockSpec((2, HID), lambda i: (0, 0)),
            pl.BlockSpec((1, HID), lambda i: (0, 0)),
            pl.BlockSpec((HID, HID), lambda i: (0, 0)),
            pl.BlockSpec((1, HID), lambda i: (0, 0)),
            pl.BlockSpec((NB, HID), lambda i: (0, 0)),
            pl.BlockSpec((2 * HID, HID), lambda i: (0, 0)),
            pl.BlockSpec((1, HID), lambda i: (0, 0)),
            pl.BlockSpec((HID, HID), lambda i: (0, 0)),
            pl.BlockSpec((1, HID), lambda i: (0, 0)),
        ],
        out_specs=pl.BlockSpec((BE, HID), lambda i: (i, 0)),
        out_shape=jax.ShapeDtypeStruct((EPAD, HID), jnp.float32),
    )(el2, elT2, tr, tp, e_W1, e_b1, e_W2, e_b2, bond_emb, cat_W1, cat_b1,
      cat_W2, cat_b2)


def _matmul_bias_body(h_ref, W_ref, b_ref, out_ref):
    res = h_ref[:] @ W_ref[:] + b_ref[:]
    out_ref[0] = res[:, 0:HID // 2]
    out_ref[1] = res[:, HID // 2:HID]


def _node_matmul(h, W, b):
    grid = NPAD // BN
    return pl.pallas_call(
        _matmul_bias_body,
        grid=(grid,),
        in_specs=[
            pl.BlockSpec((BN, HID), lambda i: (i, 0)),
            pl.BlockSpec((HID, HID), lambda i: (0, 0)),
            pl.BlockSpec((1, HID), lambda i: (0, 0)),
        ],
        out_specs=pl.BlockSpec((2, BN, HID // 2), lambda i: (0, i, 0)),
        out_shape=jax.ShapeDtypeStruct((2, NPAD, HID // 2), jnp.float32),
    )(h, W, b)


def _node_update_body(stacked, h_ref, agg_ref, W_ref, b_ref, out_ref):
    W = W_ref[:]
    upd = agg_ref[0] @ W[0:HID // 2, :] + agg_ref[1] @ W[HID // 2:HID, :]
    res = h_ref[:] + jnp.maximum(upd + b_ref[:], 0.0)
    if stacked:
        out_ref[0] = res[:, 0:HID // 2]
        out_ref[1] = res[:, HID // 2:HID]
    else:
        out_ref[:] = res


def _node_update(h, agg, W, b, stacked=False):
    grid = NPAD // BN
    if stacked:
        out_spec = pl.BlockSpec((2, BN, HID // 2), lambda i: (0, i, 0))
        out_shape = jax.ShapeDtypeStruct((2, NPAD, HID // 2), jnp.float32)
    else:
        out_spec = pl.BlockSpec((BN, HID), lambda i: (i, 0))
        out_shape = jax.ShapeDtypeStruct((NPAD, HID), jnp.float32)
    return pl.pallas_call(
        functools.partial(_node_update_body, stacked),
        grid=(grid,),
        in_specs=[
            pl.BlockSpec((BN, HID), lambda i: (i, 0)),
            pl.BlockSpec((2, BN, HID // 2), lambda i: (0, i, 0)),
            pl.BlockSpec((HID, HID), lambda i: (0, 0)),
            pl.BlockSpec((1, HID), lambda i: (0, 0)),
        ],
        out_specs=out_spec,
        out_shape=out_shape,
    )(h, agg, W, b)


def _score_body(hh_ref, fe_ref, W1_ref, b1_ref, W2_ref, b2_ref, W3_ref,
                b3_ref, out_ref):
    W1 = W1_ref[:]
    p = jnp.maximum(hh_ref[0] @ W1[0:HID // 2, :]
                    + hh_ref[1] @ W1[HID // 2:HID, :]
                    + fe_ref[:] @ W1[HID:2 * HID, :]
                    + b1_ref[:], 0.0)
    p = jnp.maximum(p @ W2_ref[:] + b2_ref[:], 0.0)
    out_ref[:] = p @ W3_ref[:] + b3_ref[:]


def _score(hh, fedge, s_W1, s_b1, s_W2, s_b2, s_W3, s_b3):
    grid = EPAD // BE
    return pl.pallas_call(
        _score_body,
        grid=(grid,),
        in_specs=[
            pl.BlockSpec((2, BE, HID // 2), lambda i: (0, i, 0)),
            pl.BlockSpec((BE, HID), lambda i: (i, 0)),
            pl.BlockSpec((2 * HID, HID), lambda i: (0, 0)),
            pl.BlockSpec((1, HID), lambda i: (0, 0)),
            pl.BlockSpec((HID, HID // 2), lambda i: (0, 0)),
            pl.BlockSpec((1, HID // 2), lambda i: (0, 0)),
            pl.BlockSpec((HID // 2, 1), lambda i: (0, 0)),
            pl.BlockSpec((1, 1), lambda i: (0, 0)),
        ],
        out_specs=pl.BlockSpec((BE, 1), lambda i: (i, 0)),
        out_shape=jax.ShapeDtypeStruct((EPAD, 1), jnp.float32),
    )(hh, fedge, s_W1, s_b1, s_W2, s_b2, s_W3, s_b3)


_HH = HID // 2
_ROWS_PER_TILE = 632
_AGG_ROWS = 16 * _ROWS_PER_TILE
_TRASH = _AGG_ROWS - 1
_EPT = EPAD // 16
_PCK = 128
_NPC = _EPT // _PCK
_CCK = 64
_CNC = _EPT // _CCK


def _mul_rows(a, b, o, nrows):
    @pl.loop(0, nrows)
    def _(r):
        for cg in range(_HH // 16):
            sl = pl.ds(cg * 16, 16)
            o[r, sl] = a[r, sl] * b[r, sl]


@functools.lru_cache(maxsize=None)
def _make_conv_sc(conv_i):
    I = conv_i

    @functools.partial(
        pl.kernel,
        mesh=plsc.VectorSubcoreMesh(core_axis_name="c", subcore_axis_name="s"),
        out_type=jax.ShapeDtypeStruct((2 * NPAD, _HH), jnp.float32),
        scratch_types=[
            pltpu.VMEM((4, _CCK), jnp.int32),
            pltpu.VMEM((4, _CCK), jnp.int32),
            [pltpu.VMEM((_CCK, _HH), jnp.float32) for _ in range(2)],
            [pltpu.VMEM((_CCK, _HH), jnp.float32) for _ in range(2)],
            [pltpu.VMEM((_CCK, _HH), jnp.float32) for _ in range(2)],
            pltpu.VMEM_SHARED((_AGG_ROWS, _HH), jnp.float32),
            [pltpu.SemaphoreType.DMA for _ in range(8)],
        ],
    )
    def conv_sc(hn2, en, src_r, dst_r, zrows, out, srcv, dstv, gath, enb,
                sbuf, aggs, sems):
        c = lax.axis_index("c")
        s = lax.axis_index("s")
        si = sems[0:2]
        sg = sems[2:4]
        se = sems[4:6]
        ss = sems[6:8]
        row0 = c * EPAD + s * _EPT
        dummy = en.at[I, pl.ds(0, _CCK), :]
        dummy_i = src_r.at[c, s, 0]

        def start_idx(k, b):
            slot = lax.rem(k, 4)
            pltpu.async_copy(src_r.at[c, s, k], srcv.at[slot], si[b])
            pltpu.async_copy(dst_r.at[s, k], dstv.at[slot], si[b])

        def wait_idx(b):
            pltpu.make_async_copy(dummy_i, srcv.at[0], si[b]).wait()
            pltpu.make_async_copy(dummy_i, dstv.at[0], si[b]).wait()

        def start_ge(k, b):
            slot = lax.rem(k, 4)
            pltpu.async_copy(hn2.at[srcv.at[slot]], gath[b], sg[b])
            pltpu.async_copy(en.at[I, pl.ds(row0 + k * _CCK, _CCK), :],
                             enb[b], se[b])

        start_idx(0, 0)
        start_idx(1, 1)
        wait_idx(0)
        start_ge(0, 0)

        zrow = s * _ROWS_PER_TILE
        pltpu.sync_copy(zrows, aggs.at[pl.ds(zrow, _ROWS_PER_TILE), :])
        plsc.subcore_barrier()

        @pl.loop(0, _CNC // 2)
        def _(g):
            for b in range(2):
                k = g * 2 + b

                @pl.when(k + 1 < _CNC)
                def _():
                    wait_idx(1 - b)
                    start_ge(k + 1, 1 - b)

                @pl.when(k >= 2)
                def _():
                    pltpu.make_async_copy(dummy, sbuf[b], ss[b]).wait()

                pltpu.make_async_copy(dummy, gath[b], sg[b]).wait()
                pltpu.make_async_copy(dummy, enb[b], se[b]).wait()
                _mul_rows(gath[b], enb[b], sbuf[b], _CCK)
                pltpu.async_copy(sbuf[b], aggs.at[dstv.at[lax.rem(k, 4)]],
                                 ss[b], add=True)

                @pl.when(k + 2 < _CNC)
                def _():
                    start_idx(k + 2, b)

        for b in range(2):
            pltpu.make_async_copy(dummy, sbuf[b], ss[b]).wait()
        plsc.subcore_barrier()
        pltpu.sync_copy(aggs.at[pl.ds(zrow, _ROWS_PER_TILE), :],
                        out.at[pl.ds(c * NPAD + zrow, _ROWS_PER_TILE), :])

    return conv_sc


@functools.lru_cache(maxsize=None)
def _make_pair_sc():
    @functools.partial(
        pl.kernel,
        mesh=plsc.VectorSubcoreMesh(core_axis_name="c", subcore_axis_name="s"),
        out_type=jax.ShapeDtypeStruct((2 * EPAD, _HH), jnp.float32),
        scratch_types=[
            pltpu.VMEM((_NPC, _PCK), jnp.int32),
            pltpu.VMEM((_NPC, _PCK), jnp.int32),
            [pltpu.VMEM((_PCK, _HH), jnp.float32) for _ in range(2)],
            [pltpu.VMEM((_PCK, _HH), jnp.float32) for _ in range(2)],
            [pltpu.VMEM((_PCK, _HH), jnp.float32) for _ in range(2)],
            [pltpu.SemaphoreType.DMA for _ in range(6)],
        ],
    )
    def pair_sc(h2, ia_r, ib_r, out, iav, ibv, ga, gb, sbuf, sems):
        c = lax.axis_index("c")
        s = lax.axis_index("s")
        sa = sems[0:2]
        sb = sems[2:4]
        so = sems[4:6]
        row0 = c * EPAD + s * _EPT
        dummy = h2.at[pl.ds(0, _PCK), :]

        pltpu.sync_copy(ia_r.at[c, s], iav)
        pltpu.sync_copy(ib_r.at[c, s], ibv)

        def start_fetch(k, b):
            pltpu.async_copy(h2.at[iav.at[k]], ga[b], sa[b])
            pltpu.async_copy(h2.at[ibv.at[k]], gb[b], sb[b])

        start_fetch(0, 0)
        start_fetch(1, 1)

        @pl.loop(0, _NPC // 2)
        def _(g):
            for b in range(2):
                k = g * 2 + b

                @pl.when(g >= 1)
                def _():
                    pltpu.make_async_copy(dummy, sbuf[b], so[b]).wait()

                pltpu.make_async_copy(dummy, ga[b], sa[b]).wait()
                pltpu.make_async_copy(dummy, gb[b], sb[b]).wait()
                _mul_rows(ga[b], gb[b], sbuf[b], _PCK)

                @pl.when(g < _NPC // 2 - 1)
                def _():
                    start_fetch(k + 2, b)

                pltpu.async_copy(sbuf[b],
                                 out.at[pl.ds(row0 + k * _PCK, _PCK), :],
                                 so[b])

        for b in range(2):
            pltpu.make_async_copy(dummy, sbuf[b], so[b]).wait()

    return pair_sc


_DW = (2 * EPAD) // 32
_DCK = 2048
_DG = _DCK // 16


@functools.lru_cache(maxsize=None)
def _make_dist_sc():
    @functools.partial(
        pl.kernel,
        mesh=plsc.VectorSubcoreMesh(core_axis_name="c", subcore_axis_name="s"),
        compiler_params=pltpu.CompilerParams(needs_layout_passes=False),
        out_type=[jax.ShapeDtypeStruct((2 * EPAD,), jnp.float32),
                  jax.ShapeDtypeStruct((2 * EPAD,), jnp.float32)],
        scratch_types=[
            [pltpu.VMEM((N,), jnp.float32) for _ in range(6)],
            pltpu.VMEM((_DCK,), jnp.int32),
            pltpu.VMEM((_DCK,), jnp.int32),
            pltpu.VMEM((_DCK,), jnp.float32),
            pltpu.VMEM((_DCK,), jnp.float32),
        ],
    )
    def dist_sc(px, py, pz, qx, qy, qz, i0, i1, d2a, d2b, coords, iv0, iv1,
                oa, ob):
        c = lax.axis_index("c")
        s = lax.axis_index("s")
        w = s * 2 + c
        for src_hbm, dstv in zip((px, py, pz, qx, qy, qz), coords):
            pltpu.sync_copy(src_hbm, dstv)
        base = w * _DW
        for k in range(_DW // _DCK):
            cbase = base + k * _DCK
            pltpu.sync_copy(i0.at[pl.ds(cbase, _DCK)], iv0)
            pltpu.sync_copy(i1.at[pl.ds(cbase, _DCK)], iv1)

            @pl.loop(0, _DG)
            def _(j):
                sl = pl.ds(j * 16, 16)
                a0 = iv0[sl]
                a1 = iv1[sl]
                dx = plsc.load_gather(coords[0], [a0]) - plsc.load_gather(coords[0], [a1])
                dy = plsc.load_gather(coords[1], [a0]) - plsc.load_gather(coords[1], [a1])
                dz = plsc.load_gather(coords[2], [a0]) - plsc.load_gather(coords[2], [a1])
                oa[sl] = dx * dx + dy * dy + dz * dz
                ex = plsc.load_gather(coords[3], [a0]) - plsc.load_gather(coords[3], [a1])
                ey = plsc.load_gather(coords[4], [a0]) - plsc.load_gather(coords[4], [a1])
                ez = plsc.load_gather(coords[5], [a0]) - plsc.load_gather(coords[5], [a1])
                ob[sl] = ex * ex + ey * ey + ez * ez

            pltpu.sync_copy(oa, d2a.at[pl.ds(cbase, _DCK)])
            pltpu.sync_copy(ob, d2b.at[pl.ds(cbase, _DCK)])

    return dist_sc


def _sqdist(p, i0, i1):
    d = p[i0] - p[i1]
    return jnp.sum(d * d, axis=-1, keepdims=True)


def kernel(atom_type, r_feat, p_feat, t, pos, pos_init, batch,
           current_edge_index, current_edge_feat_r, current_edge_feat_p,
           full_edge_index, full_type_r, full_type_p,
           atom_emb, atom_feat_W, bond_emb,
           e_W1, e_b1, e_W2, e_b2,
           cat_W1, cat_b1, cat_W2, cat_b2,
           in_W, in_b,
           conv_Wn, conv_bn, conv_We, conv_be, conv_Wu, conv_bu,
           s_W1, s_b1, s_W2, s_b2, s_W3, s_b3):
    pad_n = NPAD - N
    at2 = jnp.pad(atom_type.astype(jnp.int32), (0, pad_n))[:, None]
    bt2 = jnp.pad(batch.astype(jnp.int32), (0, pad_n))[:, None]
    rf2 = jnp.pad(r_feat, ((0, pad_n), (0, 0)))
    pf2 = jnp.pad(p_feat, ((0, pad_n), (0, 0)))

    h = _node_encode(at2, rf2, pf2, bt2, t[:, None], atom_emb, atom_feat_W,
                     in_W, in_b[None, :])

    def padE(x, v):
        return jnp.pad(x.reshape(16, E // 16), ((0, 0), (0, _EPT - E // 16)),
                       constant_values=v).reshape(-1)

    cei0 = padE(current_edge_index[0].astype(jnp.int32), 0)
    cei1 = padE(current_edge_index[1].astype(jnp.int32), _TRASH)
    fei0 = padE(full_edge_index[0].astype(jnp.int32), 0)
    fei1 = padE(full_edge_index[1].astype(jnp.int32), 0)

    i0cat = jnp.concatenate([cei0, fei0])
    i1cat = jnp.concatenate([jnp.where(cei1 >= N, 0, cei1), fei1])
    d2a, d2b = _make_dist_sc()(pos[:, 0], pos[:, 1], pos[:, 2],
                               pos_init[:, 0], pos_init[:, 1], pos_init[:, 2],
                               i0cat, i1cat)
    el2 = d2a[:EPAD][:, None]
    fl2 = d2a[EPAD:][:, None]
    elT2 = d2b[:EPAD][:, None]
    flT2 = d2b[EPAD:][:, None]

    en_all = _cur_edges(el2, elT2,
                        padE(current_edge_feat_r.astype(jnp.int32), 0)[:, None],
                        padE(current_edge_feat_p.astype(jnp.int32), 0)[:, None],
                        e_W1, e_b1[None, :], e_W2, e_b2[None, :], bond_emb,
                        cat_W1, cat_b1[None, :], cat_W2, cat_b2[None, :],
                        conv_We, conv_be[:, None, :])

    fedge = _full_edges(fl2, flT2,
                        padE(full_type_r.astype(jnp.int32), 0)[:, None],
                        padE(full_type_p.astype(jnp.int32), 0)[:, None],
                        e_W1, e_b1[None, :], e_W2, e_b2[None, :], bond_emb,
                        cat_W1, cat_b1[None, :], cat_W2, cat_b2[None, :])

    en_flat = en_all.reshape(NCONV, 2 * EPAD, HID // 2)
    src_r = jnp.concatenate([cei0, cei0 + NPAD]).reshape(2, 16, _CNC, _CCK)
    dst_r = cei1.reshape(16, _CNC, _CCK)
    zrows = jnp.zeros((_ROWS_PER_TILE, _HH), jnp.float32)
    for i in range(NCONV):
        hn = _node_matmul(h, conv_Wn[i], conv_bn[i][None, :])
        hn2 = hn.reshape(2 * NPAD, HID // 2)
        agg = _make_conv_sc(i)(hn2, en_flat, src_r, dst_r, zrows)
        h = _node_update(h, agg.reshape(2, NPAD, HID // 2),
                         conv_Wu[i], conv_bu[i][None, :],
                         stacked=(i == NCONV - 1))

    h2 = h.reshape(2 * NPAD, HID // 2)
    ia_r = jnp.concatenate([fei0, fei0 + NPAD]).reshape(2, 16, _NPC, _PCK)
    ib_r = jnp.concatenate([fei1, fei1 + NPAD]).reshape(2, 16, _NPC, _PCK)
    hh = _make_pair_sc()(h2, ia_r, ib_r).reshape(2, EPAD, HID // 2)
    pred = _score(hh, fedge, s_W1, s_b1[None, :], s_W2, s_b2[None, :],
                  s_W3, s_b3[None, :])
    return pred.reshape(16, _EPT)[:, :E // 16].reshape(E, 1)
